# baseline (speedup 1.0000x reference)
import functools

import jax
import jax.numpy as jnp
from jax import lax
from jax.experimental import pallas as pl
from jax.experimental.pallas import tpu as pltpu
from jax.experimental.pallas import tpu_sc as plsc

_NUM_CORES = 2
_NUM_SUBCORES = 16
_NW = _NUM_CORES * _NUM_SUBCORES
_LANES = 16
_NSLOT = 8


def _make_sc_pool(B, L, EMB, group):
    assert B % (_NW * _NSLOT * group) == 0 and EMB % _LANES == 0
    assert group * L <= 128
    bpw = B // _NW
    gpw = bpw // group
    gl = group * L
    inv_l = 1.0 / float(L)
    nvec = EMB // _LANES
    mesh = plsc.VectorSubcoreMesh(core_axis_name="c", subcore_axis_name="s")

    @functools.partial(
        pl.kernel,
        out_type=jax.ShapeDtypeStruct((B, EMB), jnp.float32),
        mesh=mesh,
        scratch_types=[
            pltpu.VMEM((gpw, gl), jnp.int32),
            pltpu.VMEM((bpw, EMB), jnp.float32),
        ]
        + [pltpu.VMEM((gl, EMB), jnp.float32) for _ in range(_NSLOT)]
        + [pltpu.SemaphoreType.DMA for _ in range(_NSLOT)],
    )
    def sc_pool(text_hbm, table_hbm, out_hbm, idx_v, out_v, *bufsems):
        bufs = bufsems[:_NSLOT]
        sems = bufsems[_NSLOT:]
        wid = lax.axis_index("c") * _NUM_SUBCORES + lax.axis_index("s")
        pltpu.sync_copy(text_hbm.at[pl.ds(wid * gpw, gpw)], idx_v)

        def accumulate(buf, g):
            for e in range(group):
                init = tuple(
                    buf[e * L, pl.ds(cb * _LANES, _LANES)] for cb in range(nvec)
                )

                def body(r, accs):
                    return tuple(
                        accs[cb] + buf[e * L + r, pl.ds(cb * _LANES, _LANES)]
                        for cb in range(nvec)
                    )

                accs = lax.fori_loop(1, L, body, init, unroll=7)
                for cb in range(nvec):
                    out_v[g * group + e, pl.ds(cb * _LANES, _LANES)] = (
                        accs[cb] * inv_l
                    )

        for k in range(_NSLOT - 1):
            pltpu.async_copy(table_hbm.at[idx_v.at[k]], bufs[k], sems[k])

        @pl.loop(0, gpw, step=_NSLOT)
        def _(j):
            for k in range(_NSLOT):
                g = j + k
                pltpu.make_async_copy(
                    table_hbm.at[idx_v.at[g]], bufs[k], sems[k]
                ).wait()

                @pl.when(g + _NSLOT - 1 < gpw)
                def _():
                    kk = (k + _NSLOT - 1) % _NSLOT
                    pltpu.async_copy(
                        table_hbm.at[idx_v.at[g + _NSLOT - 1]], bufs[kk], sems[kk]
                    )

                accumulate(bufs[k], g)

        pltpu.sync_copy(out_v, out_hbm.at[pl.ds(wid * bpw, bpw)])

    return sc_pool


def _mm_body(p_ref, w_ref, b_ref, o_ref):
    o_ref[...] = (
        lax.dot_general(
            p_ref[...],
            w_ref[...],
            (((1,), (1,)), ((), ())),
            preferred_element_type=jnp.float32,
        )
        + b_ref[...]
    )


def _make_tc_matmul(B, EMB, OUT, bm, bn):
    grid = (B // bm, pl.cdiv(OUT, bn))
    return pl.pallas_call(
        _mm_body,
        grid=grid,
        in_specs=[
            pl.BlockSpec((bm, EMB), lambda i, j: (i, 0)),
            pl.BlockSpec((bn, EMB), lambda i, j: (j, 0)),
            pl.BlockSpec((1, bn), lambda i, j: (0, j)),
        ],
        out_specs=pl.BlockSpec((bm, bn), lambda i, j: (i, j)),
        out_shape=jax.ShapeDtypeStruct((B, OUT), jnp.float32),
        compiler_params=pltpu.CompilerParams(
            dimension_semantics=("parallel", "parallel"),
        ),
    )


def kernel(text_batch, table, W, b):
    B, L = text_batch.shape
    EMB = table.shape[1]
    OUT = W.shape[0]
    group = 1
    text2 = text_batch.astype(jnp.int32).reshape(B // group, group * L)
    pooled = _make_sc_pool(B, L, EMB, group)(text2, table)
    logits = _make_tc_matmul(B, EMB, OUT, 1024, 5120)(pooled, W, b.reshape(1, OUT))
    return logits


# speedup vs baseline: 1.0163x; 1.0163x over previous
import functools

import jax
import jax.numpy as jnp
from jax import lax
from jax.experimental import pallas as pl
from jax.experimental.pallas import tpu as pltpu
from jax.experimental.pallas import tpu_sc as plsc

_NUM_CORES = 2
_NUM_SUBCORES = 16
_NW = _NUM_CORES * _NUM_SUBCORES
_LANES = 16
_NSLOT = 8


def _make_sc_pool(B, L, EMB, group):
    assert B % (_NW * _NSLOT * group) == 0 and EMB % _LANES == 0
    assert group * L <= 128
    bpw = B // _NW
    gpw = bpw // group
    gl = group * L
    inv_l = 1.0 / float(L)
    nvec = EMB // _LANES
    mesh = plsc.VectorSubcoreMesh(core_axis_name="c", subcore_axis_name="s")

    @functools.partial(
        pl.kernel,
        out_type=jax.ShapeDtypeStruct((B, EMB), jnp.float32),
        mesh=mesh,
        scratch_types=[
            pltpu.VMEM((gpw, gl), jnp.int32),
            pltpu.VMEM((bpw, EMB), jnp.float32),
        ]
        + [pltpu.VMEM((gl, EMB), jnp.float32) for _ in range(_NSLOT)]
        + [pltpu.SemaphoreType.DMA for _ in range(_NSLOT)],
    )
    def sc_pool(text_hbm, table_hbm, out_hbm, idx_v, out_v, *bufsems):
        bufs = bufsems[:_NSLOT]
        sems = bufsems[_NSLOT:]
        wid = lax.axis_index("c") * _NUM_SUBCORES + lax.axis_index("s")
        pltpu.sync_copy(text_hbm.at[pl.ds(wid * gpw, gpw)], idx_v)

        def accumulate(buf, g):
            for e in range(group):
                init = tuple(
                    buf[e * L, pl.ds(cb * _LANES, _LANES)] for cb in range(nvec)
                )

                @plsc.parallel_loop(1, L, step=1, unroll=2, carry=init)
                def accs(r, acc):
                    return tuple(
                        acc[cb] + buf[e * L + r, pl.ds(cb * _LANES, _LANES)]
                        for cb in range(nvec)
                    )

                for cb in range(nvec):
                    out_v[g * group + e, pl.ds(cb * _LANES, _LANES)] = (
                        accs[cb] * inv_l
                    )

        for k in range(_NSLOT - 1):
            pltpu.async_copy(table_hbm.at[idx_v.at[k]], bufs[k], sems[k])

        @pl.loop(0, gpw, step=_NSLOT)
        def _(j):
            for k in range(_NSLOT):
                g = j + k
                pltpu.make_async_copy(
                    table_hbm.at[idx_v.at[g]], bufs[k], sems[k]
                ).wait()

                @pl.when(g + _NSLOT - 1 < gpw)
                def _():
                    kk = (k + _NSLOT - 1) % _NSLOT
                    pltpu.async_copy(
                        table_hbm.at[idx_v.at[g + _NSLOT - 1]], bufs[kk], sems[kk]
                    )

                accumulate(bufs[k], g)

        pltpu.sync_copy(out_v, out_hbm.at[pl.ds(wid * bpw, bpw)])

    return sc_pool


def _mm_body(p_ref, w_ref, b_ref, o_ref):
    o_ref[...] = (
        lax.dot_general(
            p_ref[...],
            w_ref[...],
            (((1,), (1,)), ((), ())),
            preferred_element_type=jnp.float32,
        )
        + b_ref[...]
    )


def _make_tc_matmul(B, EMB, OUT, bm, bn):
    grid = (B // bm, pl.cdiv(OUT, bn))
    return pl.pallas_call(
        _mm_body,
        grid=grid,
        in_specs=[
            pl.BlockSpec((bm, EMB), lambda i, j: (i, 0)),
            pl.BlockSpec((bn, EMB), lambda i, j: (j, 0)),
            pl.BlockSpec((1, bn), lambda i, j: (0, j)),
        ],
        out_specs=pl.BlockSpec((bm, bn), lambda i, j: (i, j)),
        out_shape=jax.ShapeDtypeStruct((B, OUT), jnp.float32),
        compiler_params=pltpu.CompilerParams(
            dimension_semantics=("parallel", "parallel"),
        ),
    )


def kernel(text_batch, table, W, b):
    B, L = text_batch.shape
    EMB = table.shape[1]
    OUT = W.shape[0]
    group = 1
    text2 = text_batch.astype(jnp.int32).reshape(B // group, group * L)
    pooled = _make_sc_pool(B, L, EMB, group)(text2, table)
    logits = _make_tc_matmul(B, EMB, OUT, 2048, 2048)(pooled, W, b.reshape(1, OUT))
    return logits
